# baseline (speedup 1.0000x reference)
import functools

import jax
import jax.numpy as jnp
from jax.experimental import pallas as pl
from jax.experimental.pallas import tpu as pltpu

NC = 19
H = W = 512
B = 4
NPIX = H * W
PBLK = 32768
NBLK = NPIX // PBLK
DEN = 1e-08


def _ow_body(logits_ref, gt_ref, out_ref, acc):
    step = pl.program_id(0) * NBLK + pl.program_id(1)
    nsteps = B * NBLK

    lp = logits_ref[0]
    gt = gt_ref[0]

    cls = jax.lax.broadcasted_iota(jnp.int32, (NC, PBLK), 0)
    m = jnp.max(lp, axis=0, keepdims=True)
    q = jnp.sum(lp * lp, axis=0, keepdims=True)

    onehot = jnp.where(gt == cls, 1.0, 0.0)
    tp = jnp.where(lp >= m, onehot, 0.0)

    aug = jnp.concatenate(
        [lp, q, jnp.ones((1, PBLK), jnp.float32)], axis=0
    )
    left = jnp.concatenate([onehot, tp], axis=0)

    dims = (((1,), (1,)), ((), ()))
    stats = jax.lax.dot_general(left, aug, dims,
                                preferred_element_type=jnp.float32)

    @pl.when(step == 0)
    def _init():
        acc[...] = stats

    @pl.when(step != 0)
    def _accum():
        acc[...] += stats

    @pl.when(step == nsteps - 1)
    def _finish():
        a = acc[:NC]
        t = acc[NC:]
        s_mat = a[:, :NC]
        q_col = a[:, NC:NC + 1]
        n_col = a[:, NC + 1:NC + 2]
        stp_mat = t[:, :NC]
        ntp_col = t[:, NC + 1:NC + 2]

        has_tp = ntp_col > 0.0
        mav = jnp.where(has_tp, stp_mat / jnp.maximum(ntp_col, 1.0), 0.0)
        cross = jnp.sum(mav * s_mat, axis=1, keepdims=True)
        mavsq = jnp.sum(mav * mav, axis=1, keepdims=True)
        sq = q_col - 2.0 * cross + n_col * mavsq
        term = sq / (jnp.maximum(n_col, 1.0) * float(NC)) / DEN

        labels = jax.lax.broadcasted_iota(jnp.int32, (NC, 1), 0)
        present = n_col > 0.0
        max_present = jnp.max(jnp.where(present, labels, -1))
        include = present & (labels != max_present) & has_tp
        out_ref[...] = jnp.sum(jnp.where(include, term, 0.0),
                               axis=0, keepdims=True)


@functools.partial(jax.jit, static_argnames=())
def _ow_loss_pallas(logits, sem_gt):
    logits3 = logits.reshape(B, NC, NPIX)
    gt3 = sem_gt.reshape(B * NBLK, 1, PBLK)
    out = pl.pallas_call(
        _ow_body,
        grid=(B, NBLK),
        in_specs=[
            pl.BlockSpec((1, NC, PBLK), lambda b, j: (b, 0, j)),
            pl.BlockSpec((1, 1, PBLK), lambda b, j: (b * NBLK + j, 0, 0)),
        ],
        out_specs=pl.BlockSpec((1, 1), lambda b, j: (0, 0)),
        out_shape=jax.ShapeDtypeStruct((1, 1), jnp.float32),
        scratch_shapes=[
            pltpu.VMEM((2 * NC, NC + 2), jnp.float32),
        ],
    )(logits3, gt3)
    return out[0, 0]


def kernel(logits, sem_gt, is_train):
    loss = _ow_loss_pallas(logits, sem_gt)
    return jnp.where(is_train != 0, loss, jnp.array(0.0, jnp.float32))


# speedup vs baseline: 1.0193x; 1.0193x over previous
import functools

import jax
import jax.numpy as jnp
from jax.experimental import pallas as pl
from jax.experimental.pallas import tpu as pltpu

NC = 19
H = W = 512
B = 4
NPIX = H * W
PBLK = 32768
NBLK = NPIX // PBLK
DEN = 1e-08


def _ow_body(logits_ref, gt_ref, out_ref, acc):
    step = pl.program_id(0) * NBLK + pl.program_id(1)
    nsteps = B * NBLK

    lp = logits_ref[0]
    gt = gt_ref[0]

    cls = jax.lax.broadcasted_iota(jnp.int32, (NC, PBLK), 0)
    m = jnp.max(lp, axis=0, keepdims=True)
    q = jnp.sum(lp * lp, axis=0, keepdims=True)

    onehot = jnp.where(gt == cls, 1.0, 0.0)
    tp = jnp.where(lp >= m, onehot, 0.0)

    aug = jnp.concatenate(
        [lp, q, jnp.ones((1, PBLK), jnp.float32)], axis=0
    ).astype(jnp.bfloat16)

    dims = (((1,), (1,)), ((), ()))
    oh_bf = onehot.astype(jnp.bfloat16)
    tp_bf = tp.astype(jnp.bfloat16)
    s_all = jax.lax.dot_general(oh_bf, aug, dims,
                                preferred_element_type=jnp.float32)
    s_tp = jax.lax.dot_general(tp_bf, aug, dims,
                               preferred_element_type=jnp.float32)

    @pl.when(step == 0)
    def _init():
        acc[:NC] = s_all
        acc[NC:] = s_tp

    @pl.when(step != 0)
    def _accum():
        acc[:NC] += s_all
        acc[NC:] += s_tp

    @pl.when(step == nsteps - 1)
    def _finish():
        a = acc[:NC]
        t = acc[NC:]
        s_mat = a[:, :NC]
        q_col = a[:, NC:NC + 1]
        n_col = a[:, NC + 1:NC + 2]
        stp_mat = t[:, :NC]
        ntp_col = t[:, NC + 1:NC + 2]

        has_tp = ntp_col > 0.0
        mav = jnp.where(has_tp, stp_mat / jnp.maximum(ntp_col, 1.0), 0.0)
        cross = jnp.sum(mav * s_mat, axis=1, keepdims=True)
        mavsq = jnp.sum(mav * mav, axis=1, keepdims=True)
        sq = q_col - 2.0 * cross + n_col * mavsq
        term = sq / (jnp.maximum(n_col, 1.0) * float(NC)) / DEN

        labels = jax.lax.broadcasted_iota(jnp.int32, (NC, 1), 0)
        present = n_col > 0.0
        max_present = jnp.max(jnp.where(present, labels, -1))
        include = present & (labels != max_present) & has_tp
        out_ref[...] = jnp.sum(jnp.where(include, term, 0.0),
                               axis=0, keepdims=True)


@functools.partial(jax.jit, static_argnames=())
def _ow_loss_pallas(logits, sem_gt):
    logits3 = logits.reshape(B, NC, NPIX)
    gt3 = sem_gt.reshape(B * NBLK, 1, PBLK)
    out = pl.pallas_call(
        _ow_body,
        grid=(B, NBLK),
        in_specs=[
            pl.BlockSpec((1, NC, PBLK), lambda b, j: (b, 0, j)),
            pl.BlockSpec((1, 1, PBLK), lambda b, j: (b * NBLK + j, 0, 0)),
        ],
        out_specs=pl.BlockSpec((1, 1), lambda b, j: (0, 0)),
        out_shape=jax.ShapeDtypeStruct((1, 1), jnp.float32),
        scratch_shapes=[
            pltpu.VMEM((2 * NC, NC + 2), jnp.float32),
        ],
    )(logits3, gt3)
    return out[0, 0]


def kernel(logits, sem_gt, is_train):
    loss = _ow_loss_pallas(logits, sem_gt)
    return jnp.where(is_train != 0, loss, jnp.array(0.0, jnp.float32))


# speedup vs baseline: 1.0715x; 1.0512x over previous
import functools

import jax
import jax.numpy as jnp
from jax.experimental import pallas as pl
from jax.experimental.pallas import tpu as pltpu

NC = 19
H = W = 512
B = 4
NPIX = H * W
PBLK = 65536
NBLK = NPIX // PBLK
DEN = 1e-08


def _ow_body(logits_ref, gt_ref, out_ref, acc):
    step = pl.program_id(0) * NBLK + pl.program_id(1)
    nsteps = B * NBLK

    lp = logits_ref[0]
    gt = gt_ref[0]

    cls = jax.lax.broadcasted_iota(jnp.int32, (NC, PBLK), 0)
    m = jnp.max(lp, axis=0, keepdims=True)
    q = jnp.sum(lp * lp, axis=0, keepdims=True)

    onehot = jnp.where(gt == cls, 1.0, 0.0)
    tp = jnp.where(lp >= m, onehot, 0.0)

    aug = jnp.concatenate(
        [lp, q, jnp.ones((1, PBLK), jnp.float32)], axis=0
    ).astype(jnp.bfloat16)

    dims = (((1,), (1,)), ((), ()))
    oh_bf = onehot.astype(jnp.bfloat16)
    tp_bf = tp.astype(jnp.bfloat16)
    s_all = jax.lax.dot_general(oh_bf, aug, dims,
                                preferred_element_type=jnp.float32)
    s_tp = jax.lax.dot_general(tp_bf, aug, dims,
                               preferred_element_type=jnp.float32)

    @pl.when(step == 0)
    def _init():
        acc[:NC] = s_all
        acc[NC:] = s_tp

    @pl.when(step != 0)
    def _accum():
        acc[:NC] += s_all
        acc[NC:] += s_tp

    @pl.when(step == nsteps - 1)
    def _finish():
        a = acc[:NC]
        t = acc[NC:]
        s_mat = a[:, :NC]
        q_col = a[:, NC:NC + 1]
        n_col = a[:, NC + 1:NC + 2]
        stp_mat = t[:, :NC]
        ntp_col = t[:, NC + 1:NC + 2]

        has_tp = ntp_col > 0.0
        mav = jnp.where(has_tp, stp_mat / jnp.maximum(ntp_col, 1.0), 0.0)
        cross = jnp.sum(mav * s_mat, axis=1, keepdims=True)
        mavsq = jnp.sum(mav * mav, axis=1, keepdims=True)
        sq = q_col - 2.0 * cross + n_col * mavsq
        term = sq / (jnp.maximum(n_col, 1.0) * float(NC)) / DEN

        labels = jax.lax.broadcasted_iota(jnp.int32, (NC, 1), 0)
        present = n_col > 0.0
        max_present = jnp.max(jnp.where(present, labels, -1))
        include = present & (labels != max_present) & has_tp
        out_ref[...] = jnp.sum(jnp.where(include, term, 0.0),
                               axis=0, keepdims=True)


@functools.partial(jax.jit, static_argnames=())
def _ow_loss_pallas(logits, sem_gt):
    logits3 = logits.reshape(B, NC, NPIX)
    gt3 = sem_gt.reshape(B * NBLK, 1, PBLK)
    out = pl.pallas_call(
        _ow_body,
        grid=(B, NBLK),
        in_specs=[
            pl.BlockSpec((1, NC, PBLK), lambda b, j: (b, 0, j)),
            pl.BlockSpec((1, 1, PBLK), lambda b, j: (b * NBLK + j, 0, 0)),
        ],
        out_specs=pl.BlockSpec((1, 1), lambda b, j: (0, 0)),
        out_shape=jax.ShapeDtypeStruct((1, 1), jnp.float32),
        scratch_shapes=[
            pltpu.VMEM((2 * NC, NC + 2), jnp.float32),
        ],
    )(logits3, gt3)
    return out[0, 0]


def kernel(logits, sem_gt, is_train):
    loss = _ow_loss_pallas(logits, sem_gt)
    return jnp.where(is_train != 0, loss, jnp.array(0.0, jnp.float32))
